# Initial kernel scaffold; baseline (speedup 1.0000x reference)
import functools

import jax
import jax.numpy as jnp
from jax import lax
from jax.experimental import pallas as pl
from jax.experimental.pallas import tpu as pltpu
from jax.experimental.pallas import tpu_sc as plsc

N = 10000
E = 320000
H = 8
HC = 1024
CH = 128

NC = 2
NS = 16
NW = NC * NS
EPW = E // NW
ROWS_PER_TILE = N // NS

K1 = 80
K2 = 40
NCH1 = EPW // K1
NCH2 = EPW // K2

RB = 1000
GRID = N // RB

_mesh = plsc.VectorSubcoreMesh(core_axis_name="c", subcore_axis_name="s")


def _leaky(a):
    return jnp.where(a > 0, a, 0.2 * a)


def _tc_a_body(x_ref, w_ref, as_ref, ad_ref, h_ref, s_ref, d_ref):
    h = jnp.dot(x_ref[...], w_ref[...], preferred_element_type=jnp.float32)
    h_ref[...] = h
    hr = h.reshape(RB, H, CH)
    s = jnp.sum(hr * as_ref[...][None], axis=-1)
    d = jnp.sum(hr * ad_ref[...][None], axis=-1)
    s_ref[...] = jnp.concatenate([s, s], axis=1)
    d_ref[...] = jnp.concatenate([d, d], axis=1)


def _tc_a(x, W, a_s, a_d):
    return pl.pallas_call(
        _tc_a_body,
        grid=(GRID,),
        in_specs=[
            pl.BlockSpec((RB, x.shape[1]), lambda i: (i, 0)),
            pl.BlockSpec(W.shape, lambda i: (0, 0)),
            pl.BlockSpec(a_s.shape, lambda i: (0, 0)),
            pl.BlockSpec(a_d.shape, lambda i: (0, 0)),
        ],
        out_specs=[
            pl.BlockSpec((RB, HC), lambda i: (i, 0)),
            pl.BlockSpec((RB, 16), lambda i: (i, 0)),
            pl.BlockSpec((RB, 16), lambda i: (i, 0)),
        ],
        out_shape=[
            jax.ShapeDtypeStruct((N, HC), jnp.float32),
            jax.ShapeDtypeStruct((N, 16), jnp.float32),
            jax.ShapeDtypeStruct((N, 16), jnp.float32),
        ],
    )(x, W, a_s, a_d)


def _tc_a2_body(q_ref, sc_ref, b_ref, w_ref, as_ref, ad_ref,
                h_ref, s_ref, d_ref):
    q = q_ref[...]
    x = (q[0] + q[1] + sc_ref[...]) * (1.0 / H) + b_ref[...][None]
    x = jnp.maximum(x, 0.0)
    h = jnp.dot(x, w_ref[...], preferred_element_type=jnp.float32)
    h_ref[...] = h
    hr = h.reshape(RB, H, CH)
    s = jnp.sum(hr * as_ref[...][None], axis=-1)
    d = jnp.sum(hr * ad_ref[...][None], axis=-1)
    s_ref[...] = jnp.concatenate([s, s], axis=1)
    d_ref[...] = jnp.concatenate([d, d], axis=1)


def _tc_a2(q, selfc, b, W, a_s, a_d):
    return pl.pallas_call(
        _tc_a2_body,
        grid=(GRID,),
        in_specs=[
            pl.BlockSpec((2, RB, CH), lambda i: (0, i, 0)),
            pl.BlockSpec((RB, CH), lambda i: (i, 0)),
            pl.BlockSpec(b.shape, lambda i: (0,)),
            pl.BlockSpec(W.shape, lambda i: (0, 0)),
            pl.BlockSpec(a_s.shape, lambda i: (0, 0)),
            pl.BlockSpec(a_d.shape, lambda i: (0, 0)),
        ],
        out_specs=[
            pl.BlockSpec((RB, HC), lambda i: (i, 0)),
            pl.BlockSpec((RB, 16), lambda i: (i, 0)),
            pl.BlockSpec((RB, 16), lambda i: (i, 0)),
        ],
        out_shape=[
            jax.ShapeDtypeStruct((N, HC), jnp.float32),
            jax.ShapeDtypeStruct((N, 16), jnp.float32),
            jax.ShapeDtypeStruct((N, 16), jnp.float32),
        ],
    )(q, selfc, b, W, a_s, a_d)


def _tc_b_body(s_ref, d_ref, p_ref, h_ref, inv_ref, sc_ref):
    s = s_ref[...][:, :H]
    d = d_ref[...][:, :H]
    ws = jnp.exp(_leaky(s + d))
    p = p_ref[...]
    denom = p[0][:, :H] + p[1][:, :H] + ws
    inv = 1.0 / denom
    inv_ref[...] = jnp.concatenate([inv, inv], axis=1)
    cs = ws * inv
    hr = h_ref[...].reshape(RB, H, CH)
    sc_ref[...] = jnp.sum(hr * cs[:, :, None], axis=1)


def _tc_b(s16, d16, dp, h):
    return pl.pallas_call(
        _tc_b_body,
        grid=(GRID,),
        in_specs=[
            pl.BlockSpec((RB, 16), lambda i: (i, 0)),
            pl.BlockSpec((RB, 16), lambda i: (i, 0)),
            pl.BlockSpec((2, RB, 16), lambda i: (0, i, 0)),
            pl.BlockSpec((RB, HC), lambda i: (i, 0)),
        ],
        out_specs=[
            pl.BlockSpec((RB, 16), lambda i: (i, 0)),
            pl.BlockSpec((RB, CH), lambda i: (i, 0)),
        ],
        out_shape=[
            jax.ShapeDtypeStruct((N, 16), jnp.float32),
            jax.ShapeDtypeStruct((N, CH), jnp.float32),
        ],
    )(s16, d16, dp, h)


def _tc_c_body(q_ref, sc_ref, b_ref, o_ref):
    q = q_ref[...]
    o_ref[...] = (q[0] + q[1] + sc_ref[...]) * (1.0 / H) + b_ref[...][None]


def _tc_c(q, selfc, b):
    return pl.pallas_call(
        _tc_c_body,
        grid=(GRID,),
        in_specs=[
            pl.BlockSpec((2, RB, CH), lambda i: (0, i, 0)),
            pl.BlockSpec((RB, CH), lambda i: (i, 0)),
            pl.BlockSpec(b.shape, lambda i: (0,)),
        ],
        out_specs=pl.BlockSpec((RB, CH), lambda i: (i, 0)),
        out_shape=jax.ShapeDtypeStruct((N, CH), jnp.float32),
    )(q, selfc, b)


@functools.partial(
    pl.kernel,
    out_type=[
        jax.ShapeDtypeStruct((E, 16), jnp.float32),
        jax.ShapeDtypeStruct((NC, N, 16), jnp.float32),
    ],
    mesh=_mesh,
    scratch_types=dict(
        idx_s=[pltpu.VMEM((K1,), jnp.int32) for _ in range(2)],
        idx_d=[pltpu.VMEM((K1,), jnp.int32) for _ in range(2)],
        sv=[pltpu.VMEM((K1, 16), jnp.float32) for _ in range(2)],
        dv=[pltpu.VMEM((K1, 16), jnp.float32) for _ in range(2)],
        wv=pltpu.VMEM((K1, 16), jnp.float32),
        acc=pltpu.VMEM_SHARED((N, 16), jnp.float32),
        sems=[pltpu.SemaphoreType.DMA for _ in range(2)],
        semd=[pltpu.SemaphoreType.DMA for _ in range(2)],
    ),
)
def _sc_pass1(src_hbm, dst_hbm, stab_hbm, dtab_hbm, z16_hbm,
              w_hbm, dp_hbm,
              idx_s, idx_d, sv, dv, wv, acc, sems, semd):
    cid = lax.axis_index("c")
    sid = lax.axis_index("s")
    base0 = (cid * NS + sid) * EPW

    pltpu.sync_copy(z16_hbm.at[pl.ds(sid * ROWS_PER_TILE, ROWS_PER_TILE)],
                    acc.at[pl.ds(sid * ROWS_PER_TILE, ROWS_PER_TILE)])
    plsc.subcore_barrier()

    def fire(chunk, b):
        base = base0 + chunk * K1
        pltpu.sync_copy(src_hbm.at[pl.ds(base, K1)], idx_s[b])
        pltpu.sync_copy(dst_hbm.at[pl.ds(base, K1)], idx_d[b])
        pltpu.async_copy(stab_hbm.at[idx_s[b]], sv[b], sems[b])
        pltpu.async_copy(dtab_hbm.at[idx_d[b]], dv[b], semd[b])

    fire(0, 0)

    def step(g, _):
        for b in range(2):
            chunk = 2 * g + b

            @pl.when(chunk + 1 < NCH1)
            def _():
                fire(chunk + 1, 1 - b)

            pltpu.make_async_copy(stab_hbm.at[idx_s[b]], sv[b], sems[b]).wait()
            pltpu.make_async_copy(dtab_hbm.at[idx_d[b]], dv[b], semd[b]).wait()

            def inner(i, _):
                wv[i] = jnp.exp(_leaky(sv[b][i] + dv[b][i]))
                return 0

            lax.fori_loop(0, K1, inner, 0, unroll=4)
            base = base0 + chunk * K1
            pltpu.sync_copy(wv, w_hbm.at[pl.ds(base, K1)])
            pltpu.sync_copy(wv, acc.at[idx_d[b]], add=True)
        return 0

    lax.fori_loop(0, NCH1 // 2, step, 0)

    plsc.subcore_barrier()
    pltpu.sync_copy(acc.at[pl.ds(sid * ROWS_PER_TILE, ROWS_PER_TILE)],
                    dp_hbm.at[cid, pl.ds(sid * ROWS_PER_TILE, ROWS_PER_TILE)])


@functools.partial(
    pl.kernel,
    out_type=jax.ShapeDtypeStruct((NC, N, CH), jnp.float32),
    mesh=_mesh,
    scratch_types=dict(
        idx_s=[pltpu.VMEM((K2,), jnp.int32) for _ in range(2)],
        idx_d=[pltpu.VMEM((K2,), jnp.int32) for _ in range(2)],
        hv=[pltpu.VMEM((K2, HC), jnp.float32) for _ in range(2)],
        wv=[pltpu.VMEM((K2, 16), jnp.float32) for _ in range(2)],
        iv=[pltpu.VMEM((K2, 16), jnp.float32) for _ in range(2)],
        cv=pltpu.VMEM((K2, 16), jnp.float32),
        mv=pltpu.VMEM((K2, CH), jnp.float32),
        acc=pltpu.VMEM_SHARED((N, CH), jnp.float32),
        semh=[pltpu.SemaphoreType.DMA for _ in range(2)],
        semi=[pltpu.SemaphoreType.DMA for _ in range(2)],
        semw=[pltpu.SemaphoreType.DMA for _ in range(2)],
    ),
)
def _sc_pass2(src_hbm, dst_hbm, w_hbm, inv_hbm, h_hbm, z128_hbm,
              out_hbm,
              idx_s, idx_d, hv, wv, iv, cv, mv, acc, semh, semi, semw):
    cid = lax.axis_index("c")
    sid = lax.axis_index("s")
    base0 = (cid * NS + sid) * EPW

    pltpu.sync_copy(z128_hbm.at[pl.ds(sid * ROWS_PER_TILE, ROWS_PER_TILE)],
                    acc.at[pl.ds(sid * ROWS_PER_TILE, ROWS_PER_TILE)])
    plsc.subcore_barrier()

    def fire(chunk, b):
        base = base0 + chunk * K2
        pltpu.sync_copy(src_hbm.at[pl.ds(base, K2)], idx_s[b])
        pltpu.sync_copy(dst_hbm.at[pl.ds(base, K2)], idx_d[b])
        pltpu.async_copy(h_hbm.at[idx_s[b]], hv[b], semh[b])
        pltpu.async_copy(inv_hbm.at[idx_d[b]], iv[b], semi[b])
        pltpu.async_copy(w_hbm.at[pl.ds(base, K2)], wv[b], semw[b])

    fire(0, 0)

    def step(g, _):
        for b in range(2):
            chunk = 2 * g + b

            @pl.when(chunk + 1 < NCH2)
            def _():
                fire(chunk + 1, 1 - b)

            base = base0 + chunk * K2
            pltpu.make_async_copy(w_hbm.at[pl.ds(base, K2)], wv[b], semw[b]).wait()
            pltpu.make_async_copy(inv_hbm.at[idx_d[b]], iv[b], semi[b]).wait()

            def coef(i, _):
                cv[i] = wv[b][i] * iv[b][i]
                return 0

            lax.fori_loop(0, K2, coef, 0, unroll=4)

            pltpu.make_async_copy(h_hbm.at[idx_s[b]], hv[b], semh[b]).wait()

            def edge(e, _):
                for cb in range(H):
                    a = cv[e, 0] * hv[b][e, pl.ds(cb * 16, 16)]
                    for h in range(1, H):
                        a = a + cv[e, h] * hv[b][e, pl.ds(h * CH + cb * 16, 16)]
                    mv[e, pl.ds(cb * 16, 16)] = a
                return 0

            lax.fori_loop(0, K2, edge, 0)
            pltpu.sync_copy(mv, acc.at[idx_d[b]], add=True)
        return 0

    lax.fori_loop(0, NCH2 // 2, step, 0)

    plsc.subcore_barrier()
    pltpu.sync_copy(acc.at[pl.ds(sid * ROWS_PER_TILE, ROWS_PER_TILE)],
                    out_hbm.at[cid, pl.ds(sid * ROWS_PER_TILE, ROWS_PER_TILE)])


def kernel(x, edge_index, W1, a_s1, a_d1, b1, W2, a_s2, a_d2, b2):
    src = edge_index[0]
    dst = edge_index[1]
    z16 = jnp.zeros((N, 16), jnp.float32)
    z128 = jnp.zeros((N, CH), jnp.float32)

    h1, s16, d16 = _tc_a(x, W1, a_s1, a_d1)
    w1, dp1 = _sc_pass1(src, dst, s16, d16, z16)
    inv1, selfc1 = _tc_b(s16, d16, dp1, h1)
    q1 = _sc_pass2(src, dst, w1, inv1, h1, z128)

    h2, s16b, d16b = _tc_a2(q1, selfc1, b1, W2, a_s2, a_d2)
    w2, dp2 = _sc_pass1(src, dst, s16b, d16b, z16)
    inv2, selfc2 = _tc_b(s16b, d16b, dp2, h2)
    q2 = _sc_pass2(src, dst, w2, inv2, h2, z128)

    return _tc_c(q2, selfc2, b2)


# speedup vs baseline: 8.6701x; 8.6701x over previous
import functools

import jax
import jax.numpy as jnp
from jax import lax
from jax.experimental import pallas as pl
from jax.experimental.pallas import tpu as pltpu
from jax.experimental.pallas import tpu_sc as plsc

N = 10000
E = 320000
H = 8
HC = 1024
CH = 128

NC = 2
NS = 16
NW = NC * NS
EPW = E // NW
NPAD = 10240
ROWS_PER_TILE = NPAD // NS

K1 = 40
GOFF1 = (0, 16, 24)
K2 = 16
NCH1 = EPW // K1

RB = 1024
GRID = NPAD // RB

_mesh = plsc.VectorSubcoreMesh(core_axis_name="c", subcore_axis_name="s")


def _leaky(a):
    return jnp.where(a > 0, a, 0.2 * a)


def _flip8(m):
    return jnp.concatenate([m[:, j:j + 1] for j in range(7, -1, -1)], axis=1)


def _tc_a_body(x_ref, w_ref, as_ref, ad_ref, h_ref, sd_ref):
    h = jnp.dot(x_ref[...], w_ref[...], preferred_element_type=jnp.float32)
    h_ref[...] = jnp.concatenate(
        [h[:, k * CH + q * 32:k * CH + (q + 1) * 32]
         for q in range(4) for k in range(H)], axis=1)
    av = as_ref[...]
    dv = ad_ref[...]
    s = jnp.concatenate(
        [jnp.sum(h[:, k * CH:(k + 1) * CH] * av[k][None, :], axis=1,
                 keepdims=True) for k in range(H)], axis=1)
    d = jnp.concatenate(
        [jnp.sum(h[:, k * CH:(k + 1) * CH] * dv[k][None, :], axis=1,
                 keepdims=True) for k in range(H)], axis=1)
    sd_ref[...] = jnp.concatenate(
        [s, _flip8(d), jnp.zeros((RB, 112), jnp.float32)], axis=1)


def _tc_a(x, W, a_s, a_d):
    return pl.pallas_call(
        _tc_a_body,
        grid=(GRID,),
        in_specs=[
            pl.BlockSpec((RB, x.shape[1]), lambda i: (i, 0)),
            pl.BlockSpec(W.shape, lambda i: (0, 0)),
            pl.BlockSpec(a_s.shape, lambda i: (0, 0)),
            pl.BlockSpec(a_d.shape, lambda i: (0, 0)),
        ],
        out_specs=[
            pl.BlockSpec((RB, HC), lambda i: (i, 0)),
            pl.BlockSpec((RB, CH), lambda i: (i, 0)),
        ],
        out_shape=[
            jax.ShapeDtypeStruct((NPAD, HC), jnp.float32),
            jax.ShapeDtypeStruct((NPAD, CH), jnp.float32),
        ],
    )(x, W, a_s, a_d)


def _tc_b_body(sd_ref, p_ref, h_ref, inv_ref, sc_ref):
    sd = sd_ref[...]
    s = sd[:, :H]
    d = _flip8(sd[:, H:2 * H])
    ws = jnp.exp(_leaky(s + d))
    p = p_ref[...]
    denom = p[0][:, :H] + p[1][:, :H] + ws
    inv = 1.0 / denom
    inv_ref[...] = jnp.concatenate(
        [inv, inv, jnp.zeros((RB, 112), jnp.float32)], axis=1)
    cs = ws * inv
    hp = h_ref[...]
    sc_ref[...] = jnp.concatenate(
        [sum(hp[:, q * 256 + k * 32:q * 256 + (k + 1) * 32] * cs[:, k:k + 1]
             for k in range(H)) for q in range(4)], axis=1)


def _tc_b(sd16, dp, h):
    return pl.pallas_call(
        _tc_b_body,
        grid=(GRID,),
        in_specs=[
            pl.BlockSpec((RB, CH), lambda i: (i, 0)),
            pl.BlockSpec((2, RB, 16), lambda i: (0, i, 0)),
            pl.BlockSpec((RB, HC), lambda i: (i, 0)),
        ],
        out_specs=[
            pl.BlockSpec((RB, CH), lambda i: (i, 0)),
            pl.BlockSpec((RB, CH), lambda i: (i, 0)),
        ],
        out_shape=[
            jax.ShapeDtypeStruct((NPAD, CH), jnp.float32),
            jax.ShapeDtypeStruct((NPAD, CH), jnp.float32),
        ],
    )(sd16, dp, h)


def _tc_c_body(q1_ref, q2_ref, sc_ref, b_ref, o_ref, r_ref):
    q1 = q1_ref[...]
    q2 = q2_ref[...]
    o = (jnp.concatenate([q1[0], q1[1], q2[0], q2[1]], axis=1)
         + sc_ref[...]) * (1.0 / H) + b_ref[...][None]
    o_ref[...] = o
    r_ref[...] = jnp.maximum(o, 0.0)


def _tc_c(q1, q2, selfc, b):
    return pl.pallas_call(
        _tc_c_body,
        grid=(GRID,),
        in_specs=[
            pl.BlockSpec((2, RB, 32), lambda i: (0, i, 0)),
            pl.BlockSpec((2, RB, 32), lambda i: (0, i, 0)),
            pl.BlockSpec((RB, CH), lambda i: (i, 0)),
            pl.BlockSpec(b.shape, lambda i: (0,)),
        ],
        out_specs=[
            pl.BlockSpec((RB, CH), lambda i: (i, 0)),
            pl.BlockSpec((RB, CH), lambda i: (i, 0)),
        ],
        out_shape=[
            jax.ShapeDtypeStruct((N, CH), jnp.float32),
            jax.ShapeDtypeStruct((N, CH), jnp.float32),
        ],
    )(q1, q2, selfc, b)


NP8 = NPAD // 8
PROWS_PER_TILE = NP8 // NS


@functools.partial(
    pl.kernel,
    out_type=[
        jax.ShapeDtypeStruct((E * 16,), jnp.float32),
        jax.ShapeDtypeStruct((NC, NP8, CH), jnp.float32),
    ],
    mesh=_mesh,
    scratch_types=dict(
        idx_s=[pltpu.VMEM((K1,), jnp.int32) for _ in range(2)],
        idx_d=[pltpu.VMEM((K1,), jnp.int32) for _ in range(2)],
        sv=[pltpu.VMEM((K1, CH), jnp.float32) for _ in range(2)],
        dv=[pltpu.VMEM((K1, CH), jnp.float32) for _ in range(2)],
        idxp=pltpu.VMEM((K1,), jnp.int32),
        wf=pltpu.VMEM((K1 * 16,), jnp.float32),
        wp=pltpu.VMEM((K1, CH), jnp.float32),
        acc=pltpu.VMEM_SHARED((NP8, CH), jnp.float32),
        sems=[pltpu.SemaphoreType.DMA for _ in range(2)],
        semd=[pltpu.SemaphoreType.DMA for _ in range(2)],
    ),
)
def _sc_pass1(src_hbm, dst_hbm, sd_hbm, zp_hbm,
              w_hbm, dp_hbm,
              idx_s, idx_d, sv, dv, idxp, wf, wp, acc, sems, semd):
    cid = lax.axis_index("c")
    sid = lax.axis_index("s")
    base0 = (cid * NS + sid) * EPW
    prows = pl.ds(sid * PROWS_PER_TILE, PROWS_PER_TILE)

    pltpu.sync_copy(zp_hbm.at[prows], acc.at[prows])

    def zwp(i, _):
        wp[i // 8, pl.ds((i % 8) * 16, 16)] = jnp.zeros((16,), jnp.float32)
        return 0

    lax.fori_loop(0, K1 * 8, zwp, 0, unroll=8)
    plsc.subcore_barrier()

    def fire(chunk, b):
        base = base0 + chunk * K1
        pltpu.sync_copy(src_hbm.at[pl.ds(base, K1)], idx_s[b])
        pltpu.sync_copy(dst_hbm.at[pl.ds(base, K1)], idx_d[b])
        pltpu.async_copy(sd_hbm.at[idx_s[b]], sv[b], sems[b])
        pltpu.async_copy(sd_hbm.at[idx_d[b]], dv[b], semd[b])

    fire(0, 0)

    def step(g, _):
        for b in range(2):
            chunk = 2 * g + b

            @pl.when(chunk + 1 < NCH1)
            def _():
                fire(chunk + 1, 1 - b)

            pltpu.make_async_copy(sd_hbm.at[idx_s[b]], sv[b], sems[b]).wait()
            pltpu.make_async_copy(sd_hbm.at[idx_d[b]], dv[b], semd[b]).wait()

            for off16 in GOFF1:
                idxp[pl.ds(off16, 16)] = lax.shift_right_logical(
                    idx_d[b][pl.ds(off16, 16)], 3)

            for off16 in GOFF1:
                dvec = idx_d[b][pl.ds(off16, 16)]
                for j in range(16):
                    e = off16 + j
                    a = sv[b][e, pl.ds(0, 16)] \
                        + lax.rev(dv[b][e, pl.ds(0, 16)], (0,))
                    w = jnp.exp(_leaky(a))
                    wf[pl.ds(e * 16, 16)] = w
                    off = (dvec[j] & 7) * 16
                    wp[e, pl.ds(off, 16)] = w

            pltpu.sync_copy(wf, w_hbm.at[pl.ds((base0 + chunk * K1) * 16,
                                               K1 * 16)])
            pltpu.sync_copy(wp, acc.at[idxp], add=True)

            for off16 in GOFF1:
                dvec = idx_d[b][pl.ds(off16, 16)]
                for j in range(16):
                    off = (dvec[j] & 7) * 16
                    wp[off16 + j, pl.ds(off, 16)] = jnp.zeros(
                        (16,), jnp.float32)
        return 0

    lax.fori_loop(0, NCH1 // 2, step, 0)

    plsc.subcore_barrier()
    pltpu.sync_copy(acc.at[prows], dp_hbm.at[cid, prows])


@functools.partial(
    pl.kernel,
    out_type=jax.ShapeDtypeStruct((E * 16,), jnp.float32),
    mesh=_mesh,
    scratch_types=dict(
        idx_d=[pltpu.VMEM((K1,), jnp.int32) for _ in range(2)],
        wvb=[pltpu.VMEM((K1 * 16,), jnp.float32) for _ in range(2)],
        iv=[pltpu.VMEM((K1, CH), jnp.float32) for _ in range(2)],
        cf=pltpu.VMEM((K1 * 16,), jnp.float32),
        semi=[pltpu.SemaphoreType.DMA for _ in range(2)],
        semw=[pltpu.SemaphoreType.DMA for _ in range(2)],
    ),
)
def _sc_pass1b(dst_hbm, w_hbm, inv_hbm,
               coef_hbm,
               idx_d, wvb, iv, cf, semi, semw):
    cid = lax.axis_index("c")
    sid = lax.axis_index("s")
    base0 = (cid * NS + sid) * EPW

    def fire(chunk, b):
        base = base0 + chunk * K1
        pltpu.sync_copy(dst_hbm.at[pl.ds(base, K1)], idx_d[b])
        pltpu.async_copy(inv_hbm.at[idx_d[b]], iv[b], semi[b])
        pltpu.async_copy(w_hbm.at[pl.ds(base * 16, K1 * 16)], wvb[b], semw[b])

    fire(0, 0)

    def step(g, _):
        for b in range(2):
            chunk = 2 * g + b

            @pl.when(chunk + 1 < NCH1)
            def _():
                fire(chunk + 1, 1 - b)

            base = base0 + chunk * K1
            pltpu.make_async_copy(w_hbm.at[pl.ds(base * 16, K1 * 16)], wvb[b],
                                  semw[b]).wait()
            pltpu.make_async_copy(inv_hbm.at[idx_d[b]], iv[b], semi[b]).wait()

            def coef(i, _):
                cf[pl.ds(i * 16, 16)] = (wvb[b][pl.ds(i * 16, 16)]
                                         * iv[b][i, pl.ds(0, 16)])
                return 0

            lax.fori_loop(0, K1, coef, 0, unroll=4)
            pltpu.sync_copy(cf, coef_hbm.at[pl.ds(base * 16, K1 * 16)])
        return 0

    lax.fori_loop(0, NCH1 // 2, step, 0)


EPT = E // NS
K2 = 32
NCH2 = EPT // K2
GOFF2 = (0, 16)
NPQ = NPAD // 4
QROWS_PER_TILE = NPQ // NS


def _make_pass2(phase):
    @functools.partial(
        pl.kernel,
        out_type=jax.ShapeDtypeStruct((NC, NPQ, CH), jnp.float32),
        mesh=_mesh,
        name=f"sc_pass2_{phase}",
        scratch_types=dict(
            idx_s=[pltpu.VMEM((K2,), jnp.int32) for _ in range(2)],
            idx_d=[pltpu.VMEM((K2,), jnp.int32) for _ in range(2)],
            idxh=[pltpu.VMEM((K2,), jnp.int32) for _ in range(2)],
            idxk=pltpu.VMEM((K2,), jnp.int32),
            hv=[pltpu.VMEM((K2, 256), jnp.float32) for _ in range(2)],
            cfv=[pltpu.VMEM((K2 * 16,), jnp.float32) for _ in range(2)],
            mv=pltpu.VMEM((K2, CH), jnp.float32),
            acc=pltpu.VMEM_SHARED((NPQ, CH), jnp.float32),
            semh=[pltpu.SemaphoreType.DMA for _ in range(2)],
            semc=[pltpu.SemaphoreType.DMA for _ in range(2)],
        ),
    )
    def _pass2(src_hbm, dst_hbm, coef_hbm, h4_hbm, zq_hbm,
               out_hbm,
               idx_s, idx_d, idxh, idxk, hv, cfv, mv, acc, semh, semc):
        cid = lax.axis_index("c")
        sid = lax.axis_index("s")
        qidx = phase * 2 + cid
        base0 = sid * EPT
        krows = pl.ds(sid * QROWS_PER_TILE, QROWS_PER_TILE)

        pltpu.sync_copy(zq_hbm.at[krows], acc.at[krows])

        def zmv(i, _):
            mv[i // 8, pl.ds((i % 8) * 16, 16)] = jnp.zeros((16,), jnp.float32)
            return 0

        lax.fori_loop(0, K2 * 8, zmv, 0, unroll=8)
        plsc.subcore_barrier()

        def fire(chunk, b):
            base = base0 + chunk * K2
            pltpu.sync_copy(src_hbm.at[pl.ds(base, K2)], idx_s[b])
            pltpu.sync_copy(dst_hbm.at[pl.ds(base, K2)], idx_d[b])

            def mkidx(i, _):
                idxh[b][pl.ds(i * 16, 16)] = (
                    idx_s[b][pl.ds(i * 16, 16)] * 4 + qidx)
                return 0

            lax.fori_loop(0, K2 // 16, mkidx, 0, unroll=K2 // 16)
            pltpu.async_copy(h4_hbm.at[idxh[b]], hv[b], semh[b])
            pltpu.async_copy(coef_hbm.at[pl.ds(base * 16, K2 * 16)], cfv[b],
                             semc[b])

        def process(chunk, b):
            base = base0 + chunk * K2
            pltpu.make_async_copy(coef_hbm.at[pl.ds(base * 16, K2 * 16)],
                                  cfv[b], semc[b]).wait()
            pltpu.make_async_copy(h4_hbm.at[idxh[b]], hv[b], semh[b]).wait()

            for off16 in GOFF2:
                dvec = idx_d[b][pl.ds(off16, 16)]
                idxk[pl.ds(off16, 16)] = lax.shift_right_logical(dvec, 2)
                for j in range(16):
                    e = off16 + j
                    cvec = cfv[b][pl.ds(e * 16, 16)]
                    off = (dvec[j] & 3) * 32
                    for cb in range(2):
                        a = cvec[0] * hv[b][e, pl.ds(cb * 16, 16)]
                        for h in range(1, H):
                            a = a + cvec[h] * hv[b][
                                e, pl.ds(h * 32 + cb * 16, 16)]
                        mv[e, pl.ds(off + cb * 16, 16)] = a

            pltpu.sync_copy(mv, acc.at[idxk], add=True)

            for off16 in GOFF2:
                dvec = idx_d[b][pl.ds(off16, 16)]
                for j in range(16):
                    off = (dvec[j] & 3) * 32
                    for cb in range(2):
                        mv[off16 + j, pl.ds(off + cb * 16, 16)] = \
                            jnp.zeros((16,), jnp.float32)

        fire(0, 0)

        def step(g, _):
            for b in range(2):
                chunk = 2 * g + b

                @pl.when(chunk + 1 < NCH2)
                def _():
                    fire(chunk + 1, 1 - b)

                process(chunk, b)
            return 0

        lax.fori_loop(0, NCH2 // 2, step, 0)
        if NCH2 % 2:
            process(NCH2 - 1, 0)

        plsc.subcore_barrier()
        pltpu.sync_copy(acc.at[krows], out_hbm.at[cid, krows])

    return _pass2


_sc_pass2a = _make_pass2(0)
_sc_pass2b = _make_pass2(1)


def kernel(x, edge_index, W1, a_s1, a_d1, b1, W2, a_s2, a_d2, b2):
    src = edge_index[0]
    dst = edge_index[1]
    zq = jnp.zeros((NPAD // 4, CH), jnp.float32)

    Ws = jnp.stack([W1, W2])
    As = jnp.stack([a_s1, a_s2])
    Ad = jnp.stack([a_d1, a_d2])
    Bs = jnp.stack([b1, b2])

    def layer_step(xc, wts):
        W, a_s, a_d, bvec = wts
        h, sd16 = _tc_a(xc, W, a_s, a_d)
        w, dp = _sc_pass1(src, dst, sd16, zq)
        inv128, selfc = _tc_b(sd16, dp.reshape(NC, NPAD, 16), h)
        coef = _sc_pass1b(dst, w, inv128)
        h4 = h.reshape(NPAD * 4, 256)
        q1 = _sc_pass2a(src, dst, coef, h4, zq)
        q2 = _sc_pass2b(src, dst, coef, h4, zq)
        out, outr = _tc_c(q1.reshape(NC, NPAD, 32), q2.reshape(NC, NPAD, 32),
                          selfc, bvec)
        return outr, out

    _, ys = lax.scan(layer_step, x, (Ws, As, Ad, Bs))
    return ys[1]


# speedup vs baseline: 8.8955x; 1.0260x over previous
import functools

import jax
import jax.numpy as jnp
from jax import lax
from jax.experimental import pallas as pl
from jax.experimental.pallas import tpu as pltpu
from jax.experimental.pallas import tpu_sc as plsc

N = 10000
E = 320000
H = 8
HC = 1024
CH = 128

NC = 2
NS = 16
NW = NC * NS
EPW = E // NW
NPAD = 10240
ROWS_PER_TILE = NPAD // NS

K1 = 40
GOFF1 = (0, 16, 24)
K2 = 16
NCH1 = EPW // K1

RB = 1024
GRID = NPAD // RB

_mesh = plsc.VectorSubcoreMesh(core_axis_name="c", subcore_axis_name="s")


def _leaky(a):
    return jnp.where(a > 0, a, 0.2 * a)


def _flip8(m):
    return jnp.concatenate([m[:, j:j + 1] for j in range(7, -1, -1)], axis=1)


def _tc_a_body(x_ref, w_ref, as_ref, ad_ref, h_ref, sd_ref):
    h = jnp.dot(x_ref[...], w_ref[...], preferred_element_type=jnp.float32)
    h_ref[...] = jnp.concatenate(
        [h[:, k * CH + q * 32:k * CH + (q + 1) * 32]
         for q in range(4) for k in range(H)], axis=1)
    av = as_ref[...]
    dv = ad_ref[...]
    s = jnp.concatenate(
        [jnp.sum(h[:, k * CH:(k + 1) * CH] * av[k][None, :], axis=1,
                 keepdims=True) for k in range(H)], axis=1)
    d = jnp.concatenate(
        [jnp.sum(h[:, k * CH:(k + 1) * CH] * dv[k][None, :], axis=1,
                 keepdims=True) for k in range(H)], axis=1)
    sd_ref[...] = jnp.concatenate(
        [s, _flip8(d), jnp.zeros((RB, 112), jnp.float32)], axis=1)


def _tc_a(x, W, a_s, a_d):
    return pl.pallas_call(
        _tc_a_body,
        grid=(GRID,),
        in_specs=[
            pl.BlockSpec((RB, x.shape[1]), lambda i: (i, 0)),
            pl.BlockSpec(W.shape, lambda i: (0, 0)),
            pl.BlockSpec(a_s.shape, lambda i: (0, 0)),
            pl.BlockSpec(a_d.shape, lambda i: (0, 0)),
        ],
        out_specs=[
            pl.BlockSpec((RB, HC), lambda i: (i, 0)),
            pl.BlockSpec((RB, CH), lambda i: (i, 0)),
        ],
        out_shape=[
            jax.ShapeDtypeStruct((NPAD, HC), jnp.float32),
            jax.ShapeDtypeStruct((NPAD, CH), jnp.float32),
        ],
    )(x, W, a_s, a_d)


def _tc_b_body(sd_ref, p_ref, h_ref, inv_ref, sc_ref):
    sd = sd_ref[...]
    s = sd[:, :H]
    d = _flip8(sd[:, H:2 * H])
    ws = jnp.exp(_leaky(s + d))
    p = p_ref[...]
    denom = p[0][:, :H] + p[1][:, :H] + ws
    inv = 1.0 / denom
    inv_ref[...] = jnp.concatenate(
        [inv, inv, jnp.zeros((RB, 112), jnp.float32)], axis=1)
    cs = ws * inv
    hp = h_ref[...]
    sc_ref[...] = jnp.concatenate(
        [sum(hp[:, q * 256 + k * 32:q * 256 + (k + 1) * 32] * cs[:, k:k + 1]
             for k in range(H)) for q in range(4)], axis=1)


def _tc_b(sd16, dp, h):
    return pl.pallas_call(
        _tc_b_body,
        grid=(GRID,),
        in_specs=[
            pl.BlockSpec((RB, CH), lambda i: (i, 0)),
            pl.BlockSpec((2, RB, 16), lambda i: (0, i, 0)),
            pl.BlockSpec((RB, HC), lambda i: (i, 0)),
        ],
        out_specs=[
            pl.BlockSpec((RB, CH), lambda i: (i, 0)),
            pl.BlockSpec((RB, CH), lambda i: (i, 0)),
        ],
        out_shape=[
            jax.ShapeDtypeStruct((NPAD, CH), jnp.float32),
            jax.ShapeDtypeStruct((NPAD, CH), jnp.float32),
        ],
    )(sd16, dp, h)


def _tc_c_body(q1_ref, q2_ref, sc_ref, b_ref, o_ref, r_ref):
    q1 = q1_ref[...]
    q2 = q2_ref[...]
    o = (jnp.concatenate([q1[0], q1[1], q2[0], q2[1]], axis=1)
         + sc_ref[...]) * (1.0 / H) + b_ref[...][None]
    o_ref[...] = o
    r_ref[...] = jnp.maximum(o, 0.0)


def _tc_c(q1, q2, selfc, b):
    return pl.pallas_call(
        _tc_c_body,
        grid=(GRID,),
        in_specs=[
            pl.BlockSpec((2, RB, 32), lambda i: (0, i, 0)),
            pl.BlockSpec((2, RB, 32), lambda i: (0, i, 0)),
            pl.BlockSpec((RB, CH), lambda i: (i, 0)),
            pl.BlockSpec(b.shape, lambda i: (0,)),
        ],
        out_specs=[
            pl.BlockSpec((RB, CH), lambda i: (i, 0)),
            pl.BlockSpec((RB, CH), lambda i: (i, 0)),
        ],
        out_shape=[
            jax.ShapeDtypeStruct((N, CH), jnp.float32),
            jax.ShapeDtypeStruct((N, CH), jnp.float32),
        ],
    )(q1, q2, selfc, b)


NP8 = NPAD // 8
PROWS_PER_TILE = NP8 // NS


@functools.partial(
    pl.kernel,
    out_type=[
        jax.ShapeDtypeStruct((E * 16,), jnp.float32),
        jax.ShapeDtypeStruct((NC, NP8, CH), jnp.float32),
    ],
    mesh=_mesh,
    scratch_types=dict(
        idx_s=[pltpu.VMEM((K1,), jnp.int32) for _ in range(2)],
        idx_d=[pltpu.VMEM((K1,), jnp.int32) for _ in range(2)],
        sv=[pltpu.VMEM((K1, CH), jnp.float32) for _ in range(2)],
        dv=[pltpu.VMEM((K1, CH), jnp.float32) for _ in range(2)],
        idxp=pltpu.VMEM((K1,), jnp.int32),
        wf=pltpu.VMEM((K1 * 16,), jnp.float32),
        wp=pltpu.VMEM((K1, CH), jnp.float32),
        acc=pltpu.VMEM_SHARED((NP8, CH), jnp.float32),
        sems=[pltpu.SemaphoreType.DMA for _ in range(2)],
        semd=[pltpu.SemaphoreType.DMA for _ in range(2)],
    ),
)
def _sc_pass1(src_hbm, dst_hbm, sd_hbm, zp_hbm,
              w_hbm, dp_hbm,
              idx_s, idx_d, sv, dv, idxp, wf, wp, acc, sems, semd):
    cid = lax.axis_index("c")
    sid = lax.axis_index("s")
    base0 = (cid * NS + sid) * EPW
    prows = pl.ds(sid * PROWS_PER_TILE, PROWS_PER_TILE)

    pltpu.sync_copy(zp_hbm.at[prows], acc.at[prows])

    def zwp(i, _):
        wp[i // 8, pl.ds((i % 8) * 16, 16)] = jnp.zeros((16,), jnp.float32)
        return 0

    lax.fori_loop(0, K1 * 8, zwp, 0, unroll=8)
    plsc.subcore_barrier()

    def fire(chunk, b):
        base = base0 + chunk * K1
        pltpu.sync_copy(src_hbm.at[pl.ds(base, K1)], idx_s[b])
        pltpu.sync_copy(dst_hbm.at[pl.ds(base, K1)], idx_d[b])
        pltpu.async_copy(sd_hbm.at[idx_s[b]], sv[b], sems[b])
        pltpu.async_copy(sd_hbm.at[idx_d[b]], dv[b], semd[b])

    fire(0, 0)

    def step(g, _):
        for b in range(2):
            chunk = 2 * g + b

            @pl.when(chunk + 1 < NCH1)
            def _():
                fire(chunk + 1, 1 - b)

            pltpu.make_async_copy(sd_hbm.at[idx_s[b]], sv[b], sems[b]).wait()
            pltpu.make_async_copy(sd_hbm.at[idx_d[b]], dv[b], semd[b]).wait()

            for off16 in GOFF1:
                idxp[pl.ds(off16, 16)] = lax.shift_right_logical(
                    idx_d[b][pl.ds(off16, 16)], 3)

            for off16 in GOFF1:
                dvec = idx_d[b][pl.ds(off16, 16)]
                for j in range(16):
                    e = off16 + j
                    a = sv[b][e, pl.ds(0, 16)] \
                        + lax.rev(dv[b][e, pl.ds(0, 16)], (0,))
                    w = jnp.exp(_leaky(a))
                    wf[pl.ds(e * 16, 16)] = w
                    off = (dvec[j] & 7) * 16
                    wp[e, pl.ds(off, 16)] = w

            pltpu.sync_copy(wf, w_hbm.at[pl.ds((base0 + chunk * K1) * 16,
                                               K1 * 16)])
            pltpu.sync_copy(wp, acc.at[idxp], add=True)

            for off16 in GOFF1:
                dvec = idx_d[b][pl.ds(off16, 16)]
                for j in range(16):
                    off = (dvec[j] & 7) * 16
                    wp[off16 + j, pl.ds(off, 16)] = jnp.zeros(
                        (16,), jnp.float32)
        return 0

    lax.fori_loop(0, NCH1 // 2, step, 0)

    plsc.subcore_barrier()
    pltpu.sync_copy(acc.at[prows], dp_hbm.at[cid, prows])


@functools.partial(
    pl.kernel,
    out_type=jax.ShapeDtypeStruct((E * 16,), jnp.float32),
    mesh=_mesh,
    scratch_types=dict(
        idx_d=[pltpu.VMEM((K1,), jnp.int32) for _ in range(2)],
        wvb=[pltpu.VMEM((K1 * 16,), jnp.float32) for _ in range(2)],
        iv=[pltpu.VMEM((K1, CH), jnp.float32) for _ in range(2)],
        cf=pltpu.VMEM((K1 * 16,), jnp.float32),
        semi=[pltpu.SemaphoreType.DMA for _ in range(2)],
        semw=[pltpu.SemaphoreType.DMA for _ in range(2)],
    ),
)
def _sc_pass1b(dst_hbm, w_hbm, inv_hbm,
               coef_hbm,
               idx_d, wvb, iv, cf, semi, semw):
    cid = lax.axis_index("c")
    sid = lax.axis_index("s")
    base0 = (cid * NS + sid) * EPW

    def fire(chunk, b):
        base = base0 + chunk * K1
        pltpu.sync_copy(dst_hbm.at[pl.ds(base, K1)], idx_d[b])
        pltpu.async_copy(inv_hbm.at[idx_d[b]], iv[b], semi[b])
        pltpu.async_copy(w_hbm.at[pl.ds(base * 16, K1 * 16)], wvb[b], semw[b])

    fire(0, 0)

    def step(g, _):
        for b in range(2):
            chunk = 2 * g + b

            @pl.when(chunk + 1 < NCH1)
            def _():
                fire(chunk + 1, 1 - b)

            base = base0 + chunk * K1
            pltpu.make_async_copy(w_hbm.at[pl.ds(base * 16, K1 * 16)], wvb[b],
                                  semw[b]).wait()
            pltpu.make_async_copy(inv_hbm.at[idx_d[b]], iv[b], semi[b]).wait()

            def coef(i, _):
                cf[pl.ds(i * 16, 16)] = (wvb[b][pl.ds(i * 16, 16)]
                                         * iv[b][i, pl.ds(0, 16)])
                return 0

            lax.fori_loop(0, K1, coef, 0, unroll=4)
            pltpu.sync_copy(cf, coef_hbm.at[pl.ds(base * 16, K1 * 16)])
        return 0

    lax.fori_loop(0, NCH1 // 2, step, 0)


EPT = E // NS
K2 = 16
NCH2 = EPT // K2
GOFF2 = (0,)
NPQ = NPAD // 4
QROWS_PER_TILE = NPQ // NS


def _make_pass2(phase):
    IB = 4 * K2

    @functools.partial(
        pl.kernel,
        out_type=jax.ShapeDtypeStruct((NC, NPQ, CH), jnp.float32),
        mesh=_mesh,
        name=f"sc_pass2_{phase}",
        scratch_types=dict(
            ibs=[pltpu.VMEM((IB,), jnp.int32) for _ in range(2)],
            ibd=[pltpu.VMEM((IB,), jnp.int32) for _ in range(2)],
            idxh=[pltpu.VMEM((K2,), jnp.int32) for _ in range(2)],
            idxk=[pltpu.VMEM((K2,), jnp.int32) for _ in range(2)],
            hv=[pltpu.VMEM((K2, 256), jnp.float32) for _ in range(2)],
            cfv=[pltpu.VMEM((K2 * 16,), jnp.float32) for _ in range(2)],
            mv=[pltpu.VMEM((K2, CH), jnp.float32) for _ in range(2)],
            acc=pltpu.VMEM_SHARED((NPQ, CH), jnp.float32),
            semh=[pltpu.SemaphoreType.DMA for _ in range(2)],
            semc=[pltpu.SemaphoreType.DMA for _ in range(2)],
            semm=[pltpu.SemaphoreType.DMA for _ in range(2)],
        ),
    )
    def _pass2(src_hbm, dst_hbm, coef_hbm, h4_hbm, zq_hbm,
               out_hbm,
               ibs, ibd, idxh, idxk, hv, cfv, mv, acc, semh, semc, semm):
        cid = lax.axis_index("c")
        sid = lax.axis_index("s")
        qidx = phase * 2 + cid
        base0 = sid * EPT
        krows = pl.ds(sid * QROWS_PER_TILE, QROWS_PER_TILE)

        pltpu.sync_copy(zq_hbm.at[krows], acc.at[krows])
        plsc.subcore_barrier()

        def load_batch(bg, bb):
            gbase = base0 + bg * IB
            pltpu.sync_copy(src_hbm.at[pl.ds(gbase, IB)], ibs[bb])
            pltpu.sync_copy(dst_hbm.at[pl.ds(gbase, IB)], ibd[bb])

        def fire(chunk, u, bb, b):
            base = base0 + chunk * K2
            for i in range(K2 // 16):
                idxh[b][pl.ds(i * 16, 16)] = (
                    ibs[bb][pl.ds(u * K2 + i * 16, 16)] * 4 + qidx)
            pltpu.async_copy(h4_hbm.at[idxh[b]], hv[b], semh[b])
            pltpu.async_copy(coef_hbm.at[pl.ds(base * 16, K2 * 16)], cfv[b],
                             semc[b])

        def process(chunk, u, bb, b):
            base = base0 + chunk * K2

            @pl.when(chunk >= 2)
            def _():
                pltpu.make_async_copy(mv[b], acc.at[idxk[b]], semm[b]).wait()

            pltpu.make_async_copy(coef_hbm.at[pl.ds(base * 16, K2 * 16)],
                                  cfv[b], semc[b]).wait()
            pltpu.make_async_copy(h4_hbm.at[idxh[b]], hv[b], semh[b]).wait()

            for off16 in GOFF2:
                dvec = ibd[bb][pl.ds(u * K2 + off16, 16)]
                idxk[b][pl.ds(off16, 16)] = lax.shift_right_logical(dvec, 2)
                for j in range(16):
                    e = off16 + j
                    cvec = cfv[b][pl.ds(e * 16, 16)]
                    off = (dvec[j] & 3) * 32
                    for sl in range(8):
                        mv[b][e, pl.ds(sl * 16, 16)] = jnp.zeros(
                            (16,), jnp.float32)
                    for cb in range(2):
                        a = cvec[0] * hv[b][e, pl.ds(cb * 16, 16)]
                        for h in range(1, H):
                            a = a + cvec[h] * hv[b][
                                e, pl.ds(h * 32 + cb * 16, 16)]
                        mv[b][e, pl.ds(off + cb * 16, 16)] = a

            pltpu.async_copy(mv[b], acc.at[idxk[b]], semm[b], add=True)

        load_batch(0, 0)
        fire(0, 0, 0, 0)

        def step(G, _):
            for sb in range(2):
                bg = 2 * G + sb
                for u in range(4):
                    chunk = bg * 4 + u
                    b = u % 2
                    if u < 3:
                        fire(chunk + 1, u + 1, sb, 1 - b)
                    else:
                        load_batch(bg + 1, 1 - sb)
                        fire(chunk + 1, 0, 1 - sb, 1 - b)
                    process(chunk, u, sb, b)
            return 0

        lax.fori_loop(0, NCH2 // 8, step, 0)
        fire(NCH2 - 1, 1, 0, 1)
        process(NCH2 - 2, 0, 0, 0)
        process(NCH2 - 1, 1, 0, 1)

        for b in range(2):
            pltpu.make_async_copy(mv[b], acc.at[idxk[b]], semm[b]).wait()

        plsc.subcore_barrier()
        pltpu.sync_copy(acc.at[krows], out_hbm.at[cid, krows])

    return _pass2


_sc_pass2a = _make_pass2(0)
_sc_pass2b = _make_pass2(1)


def kernel(x, edge_index, W1, a_s1, a_d1, b1, W2, a_s2, a_d2, b2):
    src = edge_index[0]
    dst = edge_index[1]
    zq = jnp.zeros((NPAD // 4, CH), jnp.float32)

    Ws = jnp.stack([W1, W2])
    As = jnp.stack([a_s1, a_s2])
    Ad = jnp.stack([a_d1, a_d2])
    Bs = jnp.stack([b1, b2])

    def layer_step(xc, wts):
        W, a_s, a_d, bvec = wts
        h, sd16 = _tc_a(xc, W, a_s, a_d)
        w, dp = _sc_pass1(src, dst, sd16, zq)
        inv128, selfc = _tc_b(sd16, dp.reshape(NC, NPAD, 16), h)
        coef = _sc_pass1b(dst, w, inv128)
        h4 = h.reshape(NPAD * 4, 256)
        q1 = _sc_pass2a(src, dst, coef, h4, zq)
        q2 = _sc_pass2b(src, dst, coef, h4, zq)
        out, outr = _tc_c(q1.reshape(NC, NPAD, 32), q2.reshape(NC, NPAD, 32),
                          selfc, bvec)
        return outr, out

    _, ys = lax.scan(layer_step, x, (Ws, As, Ad, Bs))
    return ys[1]
